# baseline (speedup 1.0000x reference)
import functools

import jax
import jax.numpy as jnp
from jax import lax
from jax.experimental import pallas as pl
from jax.experimental.pallas import tpu as pltpu
from jax.experimental.pallas import tpu_sc as plsc

NC = 2
NS = 16
NW = NC * NS
K = 128
H = 8
C = 16
HC = H * C
NEG_SLOPE = 0.2


def _nrow(N):
    align = NS * 8
    return align * (-(-(N + 1) // align))


def _proj_body(x_ref, w_ref, as_ref, ad_ref, xs_ref, al_ref, ar_ref):
    xs = jnp.dot(x_ref[...], w_ref[...], preferred_element_type=jnp.float32)
    xs_ref[...] = xs
    red = (lax.broadcasted_iota(jnp.int32, (HC, HC), 0) // C ==
           lax.broadcasted_iota(jnp.int32, (HC, HC), 1)).astype(jnp.float32)
    al_ref[...] = jnp.dot(xs * as_ref[...], red, preferred_element_type=jnp.float32)
    ar_ref[...] = jnp.dot(xs * ad_ref[...], red, preferred_element_type=jnp.float32)


def _project(x, W_src, a_src_flat, a_dst_flat):
    N = x.shape[0]
    BLK = 1000
    grid = N // BLK
    return pl.pallas_call(
        _proj_body,
        grid=(grid,),
        in_specs=[
            pl.BlockSpec((BLK, HC), lambda i: (i, 0)),
            pl.BlockSpec((HC, HC), lambda i: (0, 0)),
            pl.BlockSpec((1, HC), lambda i: (0, 0)),
            pl.BlockSpec((1, HC), lambda i: (0, 0)),
        ],
        out_specs=[
            pl.BlockSpec((BLK, HC), lambda i: (i, 0)),
            pl.BlockSpec((BLK, HC), lambda i: (i, 0)),
            pl.BlockSpec((BLK, HC), lambda i: (i, 0)),
        ],
        out_shape=[
            jax.ShapeDtypeStruct((N, HC), jnp.float32),
            jax.ShapeDtypeStruct((N, HC), jnp.float32),
            jax.ShapeDtypeStruct((N, HC), jnp.float32),
        ],
    )(x, W_src, a_src_flat, a_dst_flat)


@functools.lru_cache(maxsize=None)
def _edge_kernel(N, KCH):
    NROW = _nrow(N)
    zrows = NROW // NS
    AFL = NROW * 16
    arows = AFL // NS
    SF = K * 16
    mesh = plsc.VectorSubcoreMesh(
        core_axis_name="c", subcore_axis_name="s", num_cores=NC, num_subcores=NS)

    NH = KCH // 2

    @functools.partial(
        pl.kernel,
        out_type=(
            jax.ShapeDtypeStruct((NC, NROW, HC), jnp.float32),
            jax.ShapeDtypeStruct((NC, AFL), jnp.float32),
        ),
        mesh=mesh,
        scratch_types=[
            pltpu.VMEM((18, K), jnp.int32),
            pltpu.VMEM((1, K), jnp.int32),
            pltpu.VMEM((K, HC), jnp.float32),
            pltpu.VMEM((K, HC), jnp.float32),
            pltpu.VMEM((SF,), jnp.float32),
            pltpu.VMEM_SHARED((NROW, HC), jnp.float32),
            pltpu.VMEM_SHARED((AFL,), jnp.float32),
            pltpu.SemaphoreType.DMA,
            pltpu.SemaphoreType.DMA,
            pltpu.SemaphoreType.DMA,
            pltpu.SemaphoreType.DMA,
        ],
    )
    def k(idxc_hbm, xs_hbm, al_hbm, ar_hbm, u_out, a_out,
          ixF, dstU, b1, b2, s_v, u_sp, a_sp, semAL, semX, semA, semU):
        cid = lax.axis_index("c")
        sid = lax.axis_index("s")
        wid = sid * NC + cid

        zv16 = jnp.zeros((16,), jnp.float32)

        def zfill(t, _):
            b1[t // 8, pl.ds((t % 8) * 16, 16)] = zv16
            return 0
        lax.fori_loop(0, K * 8, zfill, 0)

        def zfill_f(i, _):
            s_v[pl.ds(i * 16, 16)] = zv16
            return 0
        lax.fori_loop(0, SF // 16, zfill_f, 0)

        zbase = zrows * sid
        nzf, nzr = zrows // K, zrows % K
        for f in range(nzf):
            pltpu.sync_copy(b1, u_sp.at[pl.ds(zbase + K * f, K)])
        if nzr:
            pltpu.sync_copy(b1.at[pl.ds(0, nzr)], u_sp.at[pl.ds(zbase + K * nzf, nzr)])

        abase = arows * sid
        naf, nar = arows // SF, arows % SF
        for f in range(naf):
            pltpu.sync_copy(s_v, a_sp.at[pl.ds(abase + SF * f, SF)])
        if nar:
            pltpu.sync_copy(s_v.at[pl.ds(0, nar)], a_sp.at[pl.ds(abase + SF * naf, nar)])

        plsc.subcore_barrier()

        def _save_dst():
            for g in range(K // 16):
                dstU[0, pl.ds(g * 16, 16)] = ixF[1, pl.ds(g * 16, 16)]

        pltpu.sync_copy(idxc_hbm.at[wid, 0], ixF)
        _save_dst()
        pltpu.async_copy(b1, u_sp.at[dstU.at[0]], semU, add=True)
        pltpu.async_copy(idxc_hbm.at[wid, 0], ixF, semAL)

        def chunk(j, _):
            pltpu.make_async_copy(idxc_hbm.at[wid, 0], ixF, semAL).wait()
            g1 = pltpu.async_copy(ar_hbm.at[ixF.at[1]], b2, semX)
            pltpu.make_async_copy(b1, u_sp.at[dstU.at[0]], semU).wait()
            _save_dst()
            g0 = pltpu.async_copy(al_hbm.at[ixF.at[0]], b1, semAL)
            g1.wait()
            g0.wait()

            def edge_s(e, _):
                a = b1[e, pl.ds(0, 16)] + b2[e, pl.ds(0, 16)]
                a = jnp.maximum(a, a * NEG_SLOPE)
                s_v[pl.ds(e * 16, 16)] = jnp.exp(a)
                return 0
            lax.fori_loop(0, K, edge_s, 0, unroll=8)

            gx = pltpu.async_copy(xs_hbm.at[ixF.at[0]], b1, semX)
            for t in range(16):
                pltpu.async_copy(s_v.at[pl.ds(t * K, K)], a_sp.at[ixF.at[2 + t]],
                                 semA, add=True)
            gx.wait()

            def edge_m(e, _):
                sv = s_v[pl.ds(e * 16, 16)]
                for h in range(H):
                    b1[e, pl.ds(h * C, C)] = b1[e, pl.ds(h * C, C)] * sv[h]
                return 0
            lax.fori_loop(0, K, edge_m, 0, unroll=4)

            for t in range(16):
                pltpu.make_async_copy(s_v.at[pl.ds(t * K, K)],
                                      a_sp.at[ixF.at[2 + t]], semA).wait()

            pltpu.async_copy(b1, u_sp.at[dstU.at[0]], semU, add=True)

            jn = jnp.minimum(j + 1, KCH - 1)
            pltpu.async_copy(idxc_hbm.at[wid, jn], ixF, semAL)
            return 0
        lax.fori_loop(0, KCH, chunk, 0)

        pltpu.make_async_copy(idxc_hbm.at[wid, 0], ixF, semAL).wait()
        pltpu.make_async_copy(b1, u_sp.at[dstU.at[0]], semU).wait()

        plsc.subcore_barrier()

        pltpu.sync_copy(u_sp.at[pl.ds(zbase, zrows)], u_out.at[cid, pl.ds(zbase, zrows)])
        pltpu.sync_copy(a_sp.at[pl.ds(abase, arows)], a_out.at[cid, pl.ds(abase, arows)])

    return k


def _combine_body(u_ref, a_ref, b_ref, o_ref):
    usum = u_ref[0] + u_ref[1]
    asum = a_ref[0] + a_ref[1]
    r = 1.0 / (asum + 1e-16)
    expand = (lax.broadcasted_iota(jnp.int32, (16, HC), 0) ==
              lax.broadcasted_iota(jnp.int32, (16, HC), 1) // C).astype(jnp.float32)
    scale = jnp.dot(r, expand, preferred_element_type=jnp.float32)
    o_ref[...] = usum * scale + b_ref[...]


def _combine_call(U, A3, bias_row, N):
    BLK = 1000
    grid = N // BLK
    return pl.pallas_call(
        _combine_body,
        grid=(grid,),
        in_specs=[
            pl.BlockSpec((NC, BLK, HC), lambda i: (0, i, 0)),
            pl.BlockSpec((NC, BLK, 16), lambda i: (0, i, 0)),
            pl.BlockSpec((1, HC), lambda i: (0, 0)),
        ],
        out_specs=pl.BlockSpec((BLK, HC), lambda i: (i, 0)),
        out_shape=jax.ShapeDtypeStruct((N, HC), jnp.float32),
    )(U, A3, bias_row)


def kernel(x, edge_index, W_src, attn_src, attn_dst, bias):
    N = x.shape[0]
    E = edge_index.shape[1]
    Et = E + N
    NROW = _nrow(N)

    xs, al_w, ar_w = _project(
        x, W_src, attn_src.reshape(1, HC), attn_dst.reshape(1, HC))
    ar_p = jnp.pad(ar_w, ((0, NROW - N), (0, 0)))

    loops = jnp.arange(N, dtype=jnp.int32)
    src = jnp.concatenate([edge_index[0].astype(jnp.int32), loops])
    dst = jnp.concatenate([edge_index[1].astype(jnp.int32), loops])

    KCH = -(-Et // (NW * K))
    KCH += KCH % 2
    pad = NW * K * KCH - Et
    src = jnp.concatenate([src, jnp.zeros((pad,), jnp.int32)])
    dst = jnp.concatenate([dst, jnp.full((pad,), N, jnp.int32)])
    src_r = src.reshape(NW, KCH, 1, K)
    dst_r = dst.reshape(NW, KCH, 1, K)
    dlx = (dst[:, None] * 16 + jnp.arange(16, dtype=jnp.int32)[None, :])
    dlx_r = dlx.reshape(NW, KCH, 16, K)
    idxc = jnp.concatenate([src_r, dst_r, dlx_r], axis=2)

    U, A = _edge_kernel(N, KCH)(idxc, xs, al_w, ar_p)
    A3 = A.reshape(NC, NROW, 16)
    return _combine_call(U, A3, bias.reshape(1, HC), N)


# speedup vs baseline: 1.5068x; 1.5068x over previous
import functools

import jax
import jax.numpy as jnp
from jax import lax
from jax.experimental import pallas as pl
from jax.experimental.pallas import tpu as pltpu
from jax.experimental.pallas import tpu_sc as plsc

NC = 2
NS = 16
NW = NC * NS
K = 128
H = 8
C = 16
HC = H * C
NEG_SLOPE = 0.2


def _nrow(N):
    align = NS * 8
    return align * (-(-(N + 1) // align))


def _proj_body(x_ref, w_ref, as_ref, ad_ref, xs_ref, al_ref, ar_ref):
    xs = jnp.dot(x_ref[...], w_ref[...], preferred_element_type=jnp.float32)
    xs_ref[...] = xs
    red = (lax.broadcasted_iota(jnp.int32, (HC, HC), 0) // C ==
           lax.broadcasted_iota(jnp.int32, (HC, HC), 1)).astype(jnp.float32)
    al_ref[...] = jnp.dot(xs * as_ref[...], red, preferred_element_type=jnp.float32)
    ar_ref[...] = jnp.dot(xs * ad_ref[...], red, preferred_element_type=jnp.float32)


def _project(x, W_src, a_src_flat, a_dst_flat):
    N = x.shape[0]
    BLK = 1000
    grid = N // BLK
    return pl.pallas_call(
        _proj_body,
        grid=(grid,),
        in_specs=[
            pl.BlockSpec((BLK, HC), lambda i: (i, 0)),
            pl.BlockSpec((HC, HC), lambda i: (0, 0)),
            pl.BlockSpec((1, HC), lambda i: (0, 0)),
            pl.BlockSpec((1, HC), lambda i: (0, 0)),
        ],
        out_specs=[
            pl.BlockSpec((BLK, HC), lambda i: (i, 0)),
            pl.BlockSpec((BLK, HC), lambda i: (i, 0)),
            pl.BlockSpec((BLK, HC), lambda i: (i, 0)),
        ],
        out_shape=[
            jax.ShapeDtypeStruct((N, HC), jnp.float32),
            jax.ShapeDtypeStruct((N, HC), jnp.float32),
            jax.ShapeDtypeStruct((N, HC), jnp.float32),
        ],
    )(x, W_src, a_src_flat, a_dst_flat)


@functools.lru_cache(maxsize=None)
def _edge_kernel(N, KCH):
    NROW = _nrow(N)
    zrows = NROW // NS
    AFL = NROW * 16
    arows = AFL // NS
    SF = K * 16
    mesh = plsc.VectorSubcoreMesh(
        core_axis_name="c", subcore_axis_name="s", num_cores=NC, num_subcores=NS)

    @functools.partial(
        pl.kernel,
        out_type=(
            jax.ShapeDtypeStruct((NC, NROW, HC), jnp.float32),
            jax.ShapeDtypeStruct((NC, AFL), jnp.float32),
        ),
        mesh=mesh,
        scratch_types=[
            pltpu.VMEM((K,), jnp.int32),
            pltpu.VMEM((1, K), jnp.int32),
            pltpu.VMEM((16, K), jnp.int32),
            pltpu.VMEM((K, HC), jnp.float32),
            pltpu.VMEM((K, HC), jnp.float32),
            pltpu.VMEM((SF,), jnp.float32),
            pltpu.VMEM_SHARED((NROW, HC), jnp.float32),
            pltpu.VMEM_SHARED((AFL,), jnp.float32),
            pltpu.SemaphoreType.DMA,
            pltpu.SemaphoreType.DMA,
        ],
    )
    def k(src_hbm, dst_hbm, dlx_hbm, xs_hbm, al_hbm, ar_hbm, u_out, a_out,
          src_v, dst_v, dlx_v, b1, b2, s_v, u_sp, a_sp, sem0, sem1):
        cid = lax.axis_index("c")
        sid = lax.axis_index("s")
        wid = sid * NC + cid

        zv16 = jnp.zeros((16,), jnp.float32)

        def zfill(t, _):
            b1[t // 8, pl.ds((t % 8) * 16, 16)] = zv16
            return 0
        lax.fori_loop(0, K * 8, zfill, 0)

        def zfill_f(i, _):
            s_v[pl.ds(i * 16, 16)] = zv16
            return 0
        lax.fori_loop(0, SF // 16, zfill_f, 0)

        zbase = zrows * sid
        nzf, nzr = zrows // K, zrows % K
        for f in range(nzf):
            pltpu.sync_copy(b1, u_sp.at[pl.ds(zbase + K * f, K)])
        if nzr:
            pltpu.sync_copy(b1.at[pl.ds(0, nzr)], u_sp.at[pl.ds(zbase + K * nzf, nzr)])

        abase = arows * sid
        naf, nar = arows // SF, arows % SF
        for f in range(naf):
            pltpu.sync_copy(s_v, a_sp.at[pl.ds(abase + SF * f, SF)])
        if nar:
            pltpu.sync_copy(s_v.at[pl.ds(0, nar)], a_sp.at[pl.ds(abase + SF * naf, nar)])

        plsc.subcore_barrier()

        def chunk(j, _):
            i0 = pltpu.async_copy(src_hbm.at[wid, j], src_v, sem0)
            i1 = pltpu.async_copy(dst_hbm.at[wid, j], dst_v, sem1)
            i2 = pltpu.async_copy(dlx_hbm.at[wid, j], dlx_v, sem0)
            i0.wait()
            i1.wait()
            i2.wait()
            g0 = pltpu.async_copy(al_hbm.at[src_v], b1, sem0)
            g1 = pltpu.async_copy(ar_hbm.at[dst_v.at[0]], b2, sem1)
            g0.wait()
            g1.wait()

            def edge_s(e, _):
                a = b1[e, pl.ds(0, 16)] + b2[e, pl.ds(0, 16)]
                a = jnp.maximum(a, a * NEG_SLOPE)
                s_v[pl.ds(e * 16, 16)] = jnp.exp(a)
                return 0
            HK = K // 2
            lax.fori_loop(0, HK, edge_s, 0, unroll=4)
            gx0 = pltpu.async_copy(xs_hbm.at[src_v.at[pl.ds(0, HK)]],
                                   b1.at[pl.ds(0, HK)], sem0)
            lax.fori_loop(HK, K, edge_s, 0, unroll=4)
            gx1 = pltpu.async_copy(xs_hbm.at[src_v.at[pl.ds(HK, HK)]],
                                   b1.at[pl.ds(HK, HK)], sem0)
            adescs = [
                pltpu.async_copy(s_v.at[pl.ds(t * K, K)], a_sp.at[dlx_v.at[t]],
                                 sem1, add=True)
                for t in range(16)
            ]
            gx0.wait()
            gx1.wait()

            def edge_m(e, _):
                sv = s_v[pl.ds(e * 16, 16)]
                for h in range(H):
                    b1[e, pl.ds(h * C, C)] = b1[e, pl.ds(h * C, C)] * sv[h]
                return 0
            lax.fori_loop(0, K, edge_m, 0, unroll=2)

            pltpu.sync_copy(b1, u_sp.at[dst_v.at[0]], add=True)
            for d in adescs:
                d.wait()
            return 0
        lax.fori_loop(0, KCH, chunk, 0)

        plsc.subcore_barrier()

        pltpu.sync_copy(u_sp.at[pl.ds(zbase, zrows)], u_out.at[cid, pl.ds(zbase, zrows)])
        pltpu.sync_copy(a_sp.at[pl.ds(abase, arows)], a_out.at[cid, pl.ds(abase, arows)])

    return k


def _combine_body(u_ref, a_ref, b_ref, o_ref):
    usum = u_ref[0] + u_ref[1]
    asum = a_ref[0] + a_ref[1]
    r = 1.0 / (asum + 1e-16)
    expand = (lax.broadcasted_iota(jnp.int32, (16, HC), 0) ==
              lax.broadcasted_iota(jnp.int32, (16, HC), 1) // C).astype(jnp.float32)
    scale = jnp.dot(r, expand, preferred_element_type=jnp.float32)
    o_ref[...] = usum * scale + b_ref[...]


def _combine_call(U, A3, bias_row, N):
    BLK = 1000
    grid = N // BLK
    return pl.pallas_call(
        _combine_body,
        grid=(grid,),
        in_specs=[
            pl.BlockSpec((NC, BLK, HC), lambda i: (0, i, 0)),
            pl.BlockSpec((NC, BLK, 16), lambda i: (0, i, 0)),
            pl.BlockSpec((1, HC), lambda i: (0, 0)),
        ],
        out_specs=pl.BlockSpec((BLK, HC), lambda i: (i, 0)),
        out_shape=jax.ShapeDtypeStruct((N, HC), jnp.float32),
    )(U, A3, bias_row)


def kernel(x, edge_index, W_src, attn_src, attn_dst, bias):
    N = x.shape[0]
    E = edge_index.shape[1]
    Et = E + N
    NROW = _nrow(N)

    xs, al_w, ar_w = _project(
        x, W_src, attn_src.reshape(1, HC), attn_dst.reshape(1, HC))
    ar_p = jnp.pad(ar_w, ((0, NROW - N), (0, 0)))

    loops = jnp.arange(N, dtype=jnp.int32)
    src = jnp.concatenate([edge_index[0].astype(jnp.int32), loops])
    dst = jnp.concatenate([edge_index[1].astype(jnp.int32), loops])

    KCH = -(-Et // (NW * K))
    pad = NW * K * KCH - Et
    src = jnp.concatenate([src, jnp.zeros((pad,), jnp.int32)])
    dst = jnp.concatenate([dst, jnp.full((pad,), N, jnp.int32)])
    src_r = src.reshape(NW, KCH, K)
    dst_r = dst.reshape(NW, KCH, 1, K)
    dlx = (dst[:, None] * 16 + jnp.arange(16, dtype=jnp.int32)[None, :])
    dlx_r = dlx.reshape(NW, KCH, 16, K)

    U, A = _edge_kernel(N, KCH)(src_r, dst_r, dlx_r, xs, al_w, ar_p)
    A3 = A.reshape(NC, NROW, 16)
    return _combine_call(U, A3, bias.reshape(1, HC), N)
